# baseline (speedup 1.0000x reference)
import functools

import jax
import jax.numpy as jnp
import numpy as np
from jax import lax
from jax.experimental import pallas as pl
from jax.experimental.pallas import tpu as pltpu
from jax.experimental.pallas import tpu_sc as plsc

B, N, D, E = 16, 2048, 256, 128
NO = N + E
NC, NS, L = 1, 16, 16
NW = NC * NS
PW = (B * E) // NW

_mesh = plsc.VectorSubcoreMesh(
    core_axis_name="c", subcore_axis_name="s", num_cores=NC, num_subcores=NS
)

_GDN = lax.GatherDimensionNumbers(
    offset_dims=(), collapsed_slice_dims=(0,), start_index_map=(0,)
)


@functools.partial(
    pl.kernel,
    out_type=jax.ShapeDtypeStruct((B * E, D), jnp.float32),
    mesh=_mesh,
    scratch_types=[
        pltpu.VMEM((E,), jnp.int32),
        pltpu.VMEM((E,), jnp.int32),
        pltpu.VMEM((E,), jnp.float32),
        pltpu.VMEM((2 * E, D), jnp.float32),
        pltpu.VMEM((E, D), jnp.float32),
        pltpu.SemaphoreType.DMA,
    ],
)
def _sc_edge_lerp(table, eidx, pos, out, g0_v, g1_v, p_v, r_v, o_v, sem):
    wid = lax.axis_index("s") * NC + lax.axis_index("c")
    base = wid * E

    pltpu.sync_copy(eidx.at[0], g0_v)
    pltpu.sync_copy(eidx.at[1], g1_v)
    pltpu.sync_copy(pos, p_v)
    off = wid * N
    for c in range(E // L):
        sl = pl.ds(c * L, L)
        g0_v[sl] = g0_v[sl] + off
        g1_v[sl] = g1_v[sl] + off
    cp0 = pltpu.async_copy(table.at[g0_v], r_v.at[pl.ds(0, E)], sem)
    cp1 = pltpu.async_copy(table.at[g1_v], r_v.at[pl.ds(E, E)], sem)
    cp0.wait()
    cp1.wait()

    def chunk(c, carry):
        pc = p_v[pl.ds(c * L, L)]

        def row(j2, carry2):
            j = c * L + j2
            pj = lax.gather(pc, jnp.full((L, 1), j2, jnp.int32), _GDN,
                            slice_sizes=(1,),
                            mode=lax.GatherScatterMode.PROMISE_IN_BOUNDS)
            qj = 1.0 - pj

            def col(k, carry3):
                sl = pl.ds(k * L, L)
                o_v[j, sl] = r_v[j, sl] * pj + r_v[E + j, sl] * qj
                return carry3

            return lax.fori_loop(0, D // L, col, carry2)

        return lax.fori_loop(0, L, row, carry)

    lax.fori_loop(0, E // L, chunk, 0)
    pltpu.sync_copy(o_v, out.at[pl.ds(base, E)])


def _tc_copy_body(in_ref, out_ref):
    out_ref[:, :N, :] = in_ref[...]


def kernel(inputs, new_pts_pos, unpool_idx):
    table = inputs.reshape(B * N, D)
    eidx = unpool_idx.astype(jnp.int32).T
    edges = _sc_edge_lerp(table, eidx, new_pts_pos)
    edges = edges.reshape(B, E, D)
    main = pl.pallas_call(
        _tc_copy_body,
        grid=(B,),
        in_specs=[pl.BlockSpec((1, N, D), lambda i: (i, 0, 0))],
        out_specs=pl.BlockSpec((1, NO, D), lambda i: (i, 0, 0)),
        out_shape=jax.ShapeDtypeStruct((B, NO, D), jnp.float32),
        compiler_params=pltpu.CompilerParams(
            dimension_semantics=("parallel",)
        ),
    )(inputs)
    return lax.dynamic_update_slice(main, edges, (0, N, 0))


# speedup vs baseline: 1.0352x; 1.0352x over previous
import functools

import jax
import jax.numpy as jnp
import numpy as np
from jax import lax
from jax.experimental import pallas as pl
from jax.experimental.pallas import tpu as pltpu
from jax.experimental.pallas import tpu_sc as plsc

B, N, D, E = 16, 2048, 256, 128
NO = N + E
NC, NS, L = 1, 16, 16
NW = NC * NS
PW = (B * E) // NW

_BOFF = jnp.asarray(np.arange(NW)[:, None, None] * N *
                    np.ones((1, 2, E), np.int64), dtype=jnp.int32)

_mesh = plsc.VectorSubcoreMesh(
    core_axis_name="c", subcore_axis_name="s", num_cores=NC, num_subcores=NS
)

_GDN = lax.GatherDimensionNumbers(
    offset_dims=(), collapsed_slice_dims=(0,), start_index_map=(0,)
)


@functools.partial(
    pl.kernel,
    out_type=jax.ShapeDtypeStruct((B * E, D), jnp.float32),
    mesh=_mesh,
    scratch_types=[
        pltpu.VMEM((E,), jnp.int32),
        pltpu.VMEM((E,), jnp.int32),
        pltpu.VMEM((E,), jnp.float32),
        pltpu.VMEM((2 * E, D), jnp.float32),
        pltpu.VMEM((E, D), jnp.float32),
        pltpu.SemaphoreType.DMA,
    ],
)
def _sc_edge_lerp(table, gidx, pos, out, g0_v, g1_v, p_v, r_v, o_v, sem):
    wid = lax.axis_index("s") * NC + lax.axis_index("c")
    base = wid * E

    pltpu.sync_copy(gidx.at[wid, 0], g0_v)
    pltpu.sync_copy(gidx.at[wid, 1], g1_v)
    pltpu.sync_copy(pos, p_v)
    cp0 = pltpu.async_copy(table.at[g0_v], r_v.at[pl.ds(0, E)], sem)
    cp1 = pltpu.async_copy(table.at[g1_v], r_v.at[pl.ds(E, E)], sem)
    cp0.wait()
    cp1.wait()

    def chunk(c, carry):
        pc = p_v[pl.ds(c * L, L)]

        def row(j2, carry2):
            j = c * L + j2
            pj = lax.gather(pc, jnp.full((L, 1), j2, jnp.int32), _GDN,
                            slice_sizes=(1,),
                            mode=lax.GatherScatterMode.PROMISE_IN_BOUNDS)
            qj = 1.0 - pj

            def col(k, carry3):
                sl = pl.ds(k * L, L)
                o_v[j, sl] = r_v[j, sl] * pj + r_v[E + j, sl] * qj
                return carry3

            return lax.fori_loop(0, D // L, col, carry2)

        return lax.fori_loop(0, L, row, carry)

    lax.fori_loop(0, E // L, chunk, 0)
    pltpu.sync_copy(o_v, out.at[pl.ds(base, E)])


def _tc_copy_body(in_ref, out_ref):
    out_ref[:, :N, :] = in_ref[...]


def kernel(inputs, new_pts_pos, unpool_idx):
    table = inputs.reshape(B * N, D)
    eidx = unpool_idx.astype(jnp.int32).T
    gidx = jnp.tile(eidx.reshape(1, 2, E), (B, 1, 1)) + _BOFF
    edges = _sc_edge_lerp(table, gidx, new_pts_pos)
    edges = edges.reshape(B, E, D)
    main = pl.pallas_call(
        _tc_copy_body,
        grid=(B,),
        in_specs=[pl.BlockSpec((1, N, D), lambda i: (i, 0, 0))],
        out_specs=pl.BlockSpec((1, NO, D), lambda i: (i, 0, 0)),
        out_shape=jax.ShapeDtypeStruct((B, NO, D), jnp.float32),
        compiler_params=pltpu.CompilerParams(
            dimension_semantics=("parallel",)
        ),
    )(inputs)
    return lax.dynamic_update_slice(main, edges, (0, N, 0))
